# Initial kernel scaffold; baseline (speedup 1.0000x reference)
import jax
import jax.numpy as jnp
from jax.experimental import pallas as pl


def kernel(x_0, x_3, edge_index_0_to_3, edge_index_3_to_0, edge_index_3_to_3, Wl_0_e03, bl_0_e03, Wr_0_e03, Wl_0_e30, bl_0_e30, Wr_0_e30, Wl_0_e33, bl_0_e33, Wr_0_e33, Wl_1_e03, bl_1_e03, Wr_1_e03, Wl_1_e30, bl_1_e30, Wr_1_e30, Wl_1_e33, bl_1_e33, Wr_1_e33, W_lin, b_lin):
    raise NotImplementedError("write your pallas kernel here")


# speedup vs baseline: 4.3237x; 4.3237x over previous
import functools

import jax
import jax.numpy as jnp
from jax import lax
from jax.experimental import pallas as pl
from jax.experimental.pallas import tpu as pltpu
from jax.experimental.pallas import tpu_sc as plsc

N = 10000
D = 128
OUT = 64
E = 160000

NC = 2
NS = 16
NW = NC * NS
EPW = E // NW
C = 125
K = EPW // C
NP = 10240
R = NP // NS

_f32 = jnp.float32


def _seg_body(n_types, with_counts, *refs):
    i = 0
    x0 = refs[i]; x3 = refs[i + 1]; i += 2
    edges = []
    for _ in range(n_types):
        edges.append((refs[i], refs[i + 1])); i += 2
    z128 = refs[i]; i += 1
    if with_counts:
        ones128 = refs[i]; i += 1
    P_outs = [refs[i + t] for t in range(n_types)]; i += n_types
    if with_counts:
        C_outs = [refs[i + t] for t in range(n_types)]; i += n_types
    sidx, didx, rows, acc, sem = refs[i:i + 5]; i += 5
    if with_counts:
        ones_v = refs[i]

    core = lax.axis_index("c")
    sub = lax.axis_index("s")
    wid = core * NS + sub
    stripe = pl.ds(sub * R, R)

    xsrc_for = [x0, x3, x3][:n_types] if n_types == 3 else [x0, x3]

    if with_counts:
        pltpu.sync_copy(ones128, ones_v)

    passes = [(xsrc_for[t], edges[t], P_outs[t], False) for t in range(n_types)]
    if with_counts:
        passes += [(None, edges[t], C_outs[t], True) for t in range(n_types)]

    for xsrc, (src_h, dst_h), out_h, is_cnt in passes:
        if not is_cnt:
            pltpu.sync_copy(src_h.at[wid], sidx)
        pltpu.sync_copy(dst_h.at[wid], didx)
        pltpu.sync_copy(z128, acc.at[stripe])
        plsc.subcore_barrier()

        if is_cnt:
            def chunk(j, carry):
                pltpu.sync_copy(ones_v, acc.at[didx.at[j]], add=True)
                return carry
        else:
            def chunk(j, carry):
                pltpu.async_copy(xsrc.at[sidx.at[j]], rows, sem).wait()
                pltpu.sync_copy(rows, acc.at[didx.at[j]], add=True)
                return carry

        lax.fori_loop(0, K, chunk, 0)
        plsc.subcore_barrier()
        out_rows = pl.ds(core * NP + sub * R, R)
        pltpu.sync_copy(acc.at[stripe], out_h.at[out_rows])


def _make_seg_kernel(n_types, with_counts=False):
    out_type = [jax.ShapeDtypeStruct((NC * NP, D), _f32) for _ in range(n_types)]
    if with_counts:
        out_type += [jax.ShapeDtypeStruct((NC * NP, D), _f32) for _ in range(n_types)]
    scratch = [
        pltpu.VMEM((K, C), jnp.int32),
        pltpu.VMEM((K, C), jnp.int32),
        pltpu.VMEM((C, D), _f32),
        pltpu.VMEM_SHARED((NP, D), _f32),
        pltpu.SemaphoreType.DMA,
    ]
    if with_counts:
        scratch.append(pltpu.VMEM((C, D), _f32))
    return pl.kernel(
        functools.partial(_seg_body, n_types, with_counts),
        out_type=out_type,
        mesh=plsc.VectorSubcoreMesh(core_axis_name="c", subcore_axis_name="s"),
        scratch_types=scratch,
    )


def _leaky(h):
    return jnp.where(h >= 0, h, 0.01 * h)


def _agg(pa, pb, ca, cb):
    cnt = jnp.maximum(ca[:, :1] + cb[:, :1], 1.0)
    return (pa + pb) / cnt


def _tc0_body(x0r, x3r, p03a, p03b, p33a, p33b, p30a, p30b,
              c03a, c03b, c33a, c33b, c30a, c30b,
              wl03, wr03, bl03, wl33, wr33, bl33, wl30, wr30, bl30,
              o0r, o3r):
    dot = functools.partial(jnp.dot, preferred_element_type=_f32)
    a03 = _agg(p03a[...], p03b[...], c03a[...], c03b[...])
    a33 = _agg(p33a[...], p33b[...], c33a[...], c33b[...])
    a30 = _agg(p30a[...], p30b[...], c30a[...], c30b[...])
    x3v = x3r[...]
    h3 = (dot(a03, wl03[...]) + bl03[...] + dot(x3v, wr03[...])
          + dot(a33, wl33[...]) + bl33[...] + dot(x3v, wr33[...]))
    h0 = dot(a30, wl30[...]) + bl30[...] + dot(x0r[...], wr30[...])
    o3r[...] = _leaky(h3)
    o0r[...] = _leaky(h0)


def _tc1_body(x3r, p03a, p03b, p33a, p33b,
              c03a, c03b, c33a, c33b,
              wl03, wr03, bl03, wl33, wr33, bl33, wlin, blin,
              outr):
    dot = functools.partial(jnp.dot, preferred_element_type=_f32)
    a03 = _agg(p03a[...], p03b[...], c03a[...], c03b[...])
    a33 = _agg(p33a[...], p33b[...], c33a[...], c33b[...])
    x3v = x3r[...]
    h3 = (dot(a03, wl03[...]) + bl03[...] + dot(x3v, wr03[...])
          + dot(a33, wl33[...]) + bl33[...] + dot(x3v, wr33[...]))
    outr[...] = dot(_leaky(h3), wlin[...]) + blin[...]


_BM = 1000


def _row_spec(w):
    return pl.BlockSpec((_BM, w), lambda i: (i, 0))


def _full_spec(shape):
    return pl.BlockSpec(shape, lambda i: (0,) * len(shape))


def _tc_layer0(x0, x3, parts, cnts, W):
    (p03a, p03b), (p33a, p33b), (p30a, p30b) = parts
    (c03a, c03b), (c33a, c33b), (c30a, c30b) = cnts
    in_specs = ([_row_spec(D)] * 14
                + [_full_spec((D, D)), _full_spec((D, D)), _full_spec((1, D))] * 3)
    out_specs = [_row_spec(D), _row_spec(D)]
    f = pl.pallas_call(
        _tc0_body,
        grid=(N // _BM,),
        in_specs=in_specs,
        out_specs=out_specs,
        out_shape=[jax.ShapeDtypeStruct((N, D), _f32)] * 2,
    )
    return f(x0, x3, p03a, p03b, p33a, p33b, p30a, p30b,
             c03a, c03b, c33a, c33b, c30a, c30b, *W)


def _tc_layer1(x3, parts, cnts, W):
    (p03a, p03b), (p33a, p33b) = parts
    (c03a, c03b), (c33a, c33b) = cnts
    in_specs = ([_row_spec(D)] * 9
                + [_full_spec((D, D)), _full_spec((D, D)), _full_spec((1, D))] * 2
                + [_full_spec((D, OUT)), _full_spec((1, OUT))])
    f = pl.pallas_call(
        _tc1_body,
        grid=(N // _BM,),
        in_specs=in_specs,
        out_specs=_row_spec(OUT),
        out_shape=jax.ShapeDtypeStruct((N, OUT), _f32),
    )
    return f(x3, p03a, p03b, p33a, p33b, c03a, c03b, c33a, c33b, *W)


def kernel(x_0, x_3, edge_index_0_to_3, edge_index_3_to_0, edge_index_3_to_3,
           Wl_0_e03, bl_0_e03, Wr_0_e03,
           Wl_0_e30, bl_0_e30, Wr_0_e30,
           Wl_0_e33, bl_0_e33, Wr_0_e33,
           Wl_1_e03, bl_1_e03, Wr_1_e03,
           Wl_1_e30, bl_1_e30, Wr_1_e30,
           Wl_1_e33, bl_1_e33, Wr_1_e33,
           W_lin, b_lin):
    s03 = edge_index_0_to_3[0].reshape(NW, K, C)
    d03 = edge_index_0_to_3[1].reshape(NW, K, C)
    s33 = edge_index_3_to_3[0].reshape(NW, K, C)
    d33 = edge_index_3_to_3[1].reshape(NW, K, C)
    s30 = edge_index_3_to_0[0].reshape(NW, K, C)
    d30 = edge_index_3_to_0[1].reshape(NW, K, C)
    z128 = jnp.zeros((R, D), _f32)
    ones128 = jnp.ones((C, D), _f32)

    seg3 = _make_seg_kernel(3, with_counts=True)
    P03, P33, P30, C03, C33, C30 = seg3(
        x_0, x_3, s03, d03, s33, d33, s30, d30, z128, ones128)

    def split(a):
        return a[:N], a[NP:NP + N]

    cn03, cn33, cn30 = split(C03), split(C33), split(C30)
    x0b, x3b = _tc_layer0(
        x_0, x_3,
        (split(P03), split(P33), split(P30)),
        (cn03, cn33, cn30),
        (Wl_0_e03, Wr_0_e03, bl_0_e03.reshape(1, D),
         Wl_0_e33, Wr_0_e33, bl_0_e33.reshape(1, D),
         Wl_0_e30, Wr_0_e30, bl_0_e30.reshape(1, D)))

    seg2 = _make_seg_kernel(2)
    Q03, Q33 = seg2(x0b, x3b, s03, d03, s33, d33, z128)

    return _tc_layer1(
        x3b, (split(Q03), split(Q33)), (cn03, cn33),
        (Wl_1_e03, Wr_1_e03, bl_1_e03.reshape(1, D),
         Wl_1_e33, Wr_1_e33, bl_1_e33.reshape(1, D),
         W_lin, b_lin.reshape(1, OUT)))


# speedup vs baseline: 5.1009x; 1.1797x over previous
import functools

import jax
import jax.numpy as jnp
from jax import lax
from jax.experimental import pallas as pl
from jax.experimental.pallas import tpu as pltpu
from jax.experimental.pallas import tpu_sc as plsc

N = 10000
D = 128
OUT = 64
E = 160000

NC = 2
NS = 16
NW = NC * NS
EPW = E // NW
C = 125
K = EPW // C
NP = 10240
R = NP // NS

_f32 = jnp.float32


def _seg_body(n_types, with_counts, *refs):
    i = 0
    x0 = refs[i]; x3 = refs[i + 1]; i += 2
    edges = []
    for _ in range(n_types):
        edges.append((refs[i], refs[i + 1])); i += 2
    z128 = refs[i]; i += 1
    if with_counts:
        ones128 = refs[i]; i += 1
    P_outs = [refs[i + t] for t in range(n_types)]; i += n_types
    if with_counts:
        C_outs = [refs[i + t] for t in range(n_types)]; i += n_types
    sidx, didx, rows_a, rows_b, acc, gsem = refs[i:i + 6]; i += 6

    core = lax.axis_index("c")
    sub = lax.axis_index("s")
    wid = core * NS + sub
    stripe = pl.ds(sub * R, R)

    xsrc_for = [x0, x3, x3][:n_types] if n_types == 3 else [x0, x3]

    passes = [(xsrc_for[t], edges[t], P_outs[t], False) for t in range(n_types)]
    if with_counts:
        passes += [(None, edges[t], C_outs[t], True) for t in range(n_types)]

    for xsrc, (src_h, dst_h), out_h, is_cnt in passes:
        if not is_cnt:
            pltpu.sync_copy(src_h.at[wid], sidx)
        pltpu.sync_copy(dst_h.at[wid], didx)
        pltpu.sync_copy(z128, acc.at[stripe])
        if is_cnt:
            pltpu.sync_copy(ones128, rows_a)
        plsc.subcore_barrier()

        if is_cnt:
            def chunk(j, carry):
                pltpu.sync_copy(rows_a, acc.at[didx.at[j]], add=True)
                return carry

            lax.fori_loop(0, K, chunk, 0)
        else:
            pltpu.async_copy(xsrc.at[sidx.at[0]], rows_a, gsem)

            def pair(j2, carry):
                for b, (cur, oth) in ((0, (rows_a, rows_b)),
                                      (1, (rows_b, rows_a))):
                    jj = 2 * j2 + b
                    pltpu.make_async_copy(
                        xsrc.at[sidx.at[jj]], cur, gsem).wait()

                    @pl.when(jj + 1 < K)
                    def _():
                        pltpu.async_copy(xsrc.at[sidx.at[jj + 1]], oth, gsem)

                    pltpu.sync_copy(cur, acc.at[didx.at[jj]], add=True)
                return carry

            lax.fori_loop(0, K // 2, pair, 0)
        plsc.subcore_barrier()
        out_rows = pl.ds(core * NP + sub * R, R)
        pltpu.sync_copy(acc.at[stripe], out_h.at[out_rows])


def _make_seg_kernel(n_types, with_counts=False):
    out_type = [jax.ShapeDtypeStruct((NC * NP, D), _f32) for _ in range(n_types)]
    if with_counts:
        out_type += [jax.ShapeDtypeStruct((NC * NP, D), _f32) for _ in range(n_types)]
    scratch = [
        pltpu.VMEM((K, C), jnp.int32),
        pltpu.VMEM((K, C), jnp.int32),
        pltpu.VMEM((C, D), _f32),
        pltpu.VMEM((C, D), _f32),
        pltpu.VMEM_SHARED((NP, D), _f32),
        pltpu.SemaphoreType.DMA,
    ]
    return pl.kernel(
        functools.partial(_seg_body, n_types, with_counts),
        out_type=out_type,
        mesh=plsc.VectorSubcoreMesh(core_axis_name="c", subcore_axis_name="s"),
        scratch_types=scratch,
    )


def _leaky(h):
    return jnp.where(h >= 0, h, 0.01 * h)


def _agg(pa, pb, ca, cb):
    cnt = jnp.maximum(ca[:, :1] + cb[:, :1], 1.0)
    return (pa + pb) / cnt


def _tc0_body(x0r, x3r, p03a, p03b, p33a, p33b, p30a, p30b,
              c03a, c03b, c33a, c33b, c30a, c30b,
              wl03, wr03, bl03, wl33, wr33, bl33, wl30, wr30, bl30,
              o0r, o3r):
    dot = functools.partial(jnp.dot, preferred_element_type=_f32)
    a03 = _agg(p03a[...], p03b[...], c03a[...], c03b[...])
    a33 = _agg(p33a[...], p33b[...], c33a[...], c33b[...])
    a30 = _agg(p30a[...], p30b[...], c30a[...], c30b[...])
    x3v = x3r[...]
    h3 = (dot(a03, wl03[...]) + bl03[...] + dot(x3v, wr03[...])
          + dot(a33, wl33[...]) + bl33[...] + dot(x3v, wr33[...]))
    h0 = dot(a30, wl30[...]) + bl30[...] + dot(x0r[...], wr30[...])
    o3r[...] = _leaky(h3)
    o0r[...] = _leaky(h0)


def _tc1_body(x3r, p03a, p03b, p33a, p33b,
              c03a, c03b, c33a, c33b,
              wl03, wr03, bl03, wl33, wr33, bl33, wlin, blin,
              outr):
    dot = functools.partial(jnp.dot, preferred_element_type=_f32)
    a03 = _agg(p03a[...], p03b[...], c03a[...], c03b[...])
    a33 = _agg(p33a[...], p33b[...], c33a[...], c33b[...])
    x3v = x3r[...]
    h3 = (dot(a03, wl03[...]) + bl03[...] + dot(x3v, wr03[...])
          + dot(a33, wl33[...]) + bl33[...] + dot(x3v, wr33[...]))
    outr[...] = dot(_leaky(h3), wlin[...]) + blin[...]


_BM = 1000


def _row_spec(w):
    return pl.BlockSpec((_BM, w), lambda i: (i, 0))


def _full_spec(shape):
    return pl.BlockSpec(shape, lambda i: (0,) * len(shape))


def _tc_layer0(x0, x3, parts, cnts, W):
    (p03a, p03b), (p33a, p33b), (p30a, p30b) = parts
    (c03a, c03b), (c33a, c33b), (c30a, c30b) = cnts
    in_specs = ([_row_spec(D)] * 14
                + [_full_spec((D, D)), _full_spec((D, D)), _full_spec((1, D))] * 3)
    out_specs = [_row_spec(D), _row_spec(D)]
    f = pl.pallas_call(
        _tc0_body,
        grid=(N // _BM,),
        in_specs=in_specs,
        out_specs=out_specs,
        out_shape=[jax.ShapeDtypeStruct((N, D), _f32)] * 2,
    )
    return f(x0, x3, p03a, p03b, p33a, p33b, p30a, p30b,
             c03a, c03b, c33a, c33b, c30a, c30b, *W)


def _tc_layer1(x3, parts, cnts, W):
    (p03a, p03b), (p33a, p33b) = parts
    (c03a, c03b), (c33a, c33b) = cnts
    in_specs = ([_row_spec(D)] * 9
                + [_full_spec((D, D)), _full_spec((D, D)), _full_spec((1, D))] * 2
                + [_full_spec((D, OUT)), _full_spec((1, OUT))])
    f = pl.pallas_call(
        _tc1_body,
        grid=(N // _BM,),
        in_specs=in_specs,
        out_specs=_row_spec(OUT),
        out_shape=jax.ShapeDtypeStruct((N, OUT), _f32),
    )
    return f(x3, p03a, p03b, p33a, p33b, c03a, c03b, c33a, c33b, *W)


def kernel(x_0, x_3, edge_index_0_to_3, edge_index_3_to_0, edge_index_3_to_3,
           Wl_0_e03, bl_0_e03, Wr_0_e03,
           Wl_0_e30, bl_0_e30, Wr_0_e30,
           Wl_0_e33, bl_0_e33, Wr_0_e33,
           Wl_1_e03, bl_1_e03, Wr_1_e03,
           Wl_1_e30, bl_1_e30, Wr_1_e30,
           Wl_1_e33, bl_1_e33, Wr_1_e33,
           W_lin, b_lin):
    s03 = edge_index_0_to_3[0].reshape(NW, K, C)
    d03 = edge_index_0_to_3[1].reshape(NW, K, C)
    s33 = edge_index_3_to_3[0].reshape(NW, K, C)
    d33 = edge_index_3_to_3[1].reshape(NW, K, C)
    s30 = edge_index_3_to_0[0].reshape(NW, K, C)
    d30 = edge_index_3_to_0[1].reshape(NW, K, C)
    z128 = jnp.zeros((R, D), _f32)
    ones128 = jnp.ones((C, D), _f32)

    seg3 = _make_seg_kernel(3, with_counts=True)
    P03, P33, P30, C03, C33, C30 = seg3(
        x_0, x_3, s03, d03, s33, d33, s30, d30, z128, ones128)

    def split(a):
        return a[:N], a[NP:NP + N]

    cn03, cn33, cn30 = split(C03), split(C33), split(C30)
    x0b, x3b = _tc_layer0(
        x_0, x_3,
        (split(P03), split(P33), split(P30)),
        (cn03, cn33, cn30),
        (Wl_0_e03, Wr_0_e03, bl_0_e03.reshape(1, D),
         Wl_0_e33, Wr_0_e33, bl_0_e33.reshape(1, D),
         Wl_0_e30, Wr_0_e30, bl_0_e30.reshape(1, D)))

    seg2 = _make_seg_kernel(2)
    Q03, Q33 = seg2(x0b, x3b, s03, d03, s33, d33, z128)

    return _tc_layer1(
        x3b, (split(Q03), split(Q33)), (cn03, cn33),
        (Wl_1_e03, Wr_1_e03, bl_1_e03.reshape(1, D),
         Wl_1_e33, Wr_1_e33, bl_1_e33.reshape(1, D),
         W_lin, b_lin.reshape(1, OUT)))


# speedup vs baseline: 5.8530x; 1.1474x over previous
import functools

import jax
import jax.numpy as jnp
from jax import lax
from jax.experimental import pallas as pl
from jax.experimental.pallas import tpu as pltpu
from jax.experimental.pallas import tpu_sc as plsc

N = 10000
D = 128
OUT = 64
E = 160000

NC = 2
NS = 16
NW = NC * NS
EPW = E // NW
C = 125
K = EPW // C
NP = 10240
R = NP // NS

_f32 = jnp.float32


def _seg_body(n_types, *refs):
    i = 0
    x0 = refs[i]; x3 = refs[i + 1]; i += 2
    edges = []
    for _ in range(n_types):
        edges.append((refs[i], refs[i + 1])); i += 2
    z128 = refs[i]; i += 1
    P_outs = [refs[i + t] for t in range(n_types)]; i += n_types
    sidx, didx, rows_a, rows_b, acc, gsem = refs[i:i + 6]; i += 6

    core = lax.axis_index("c")
    sub = lax.axis_index("s")
    wid = core * NS + sub
    stripe = pl.ds(sub * R, R)

    xsrc_for = [x0, x3, x3][:n_types] if n_types == 3 else [x0, x3]

    for t in range(n_types):
        src_h, dst_h = edges[t]
        xsrc = xsrc_for[t]
        pltpu.sync_copy(src_h.at[wid], sidx)
        pltpu.sync_copy(dst_h.at[wid], didx)
        pltpu.sync_copy(z128, acc.at[stripe])
        plsc.subcore_barrier()

        pltpu.async_copy(xsrc.at[sidx.at[0]], rows_a, gsem)

        def pair(j2, carry):
            for b, (cur, oth) in ((0, (rows_a, rows_b)),
                                  (1, (rows_b, rows_a))):
                jj = 2 * j2 + b
                pltpu.make_async_copy(
                    xsrc.at[sidx.at[jj]], cur, gsem).wait()

                @pl.when(jj + 1 < K)
                def _():
                    pltpu.async_copy(xsrc.at[sidx.at[jj + 1]], oth, gsem)

                pltpu.sync_copy(cur, acc.at[didx.at[jj]], add=True)
            return carry

        lax.fori_loop(0, K // 2, pair, 0)
        plsc.subcore_barrier()
        out_rows = pl.ds(core * NP + sub * R, R)
        pltpu.sync_copy(acc.at[stripe], P_outs[t].at[out_rows])


def _make_seg_kernel(n_types):
    out_type = [jax.ShapeDtypeStruct((NC * NP, D), _f32) for _ in range(n_types)]
    scratch = [
        pltpu.VMEM((K, C), jnp.int32),
        pltpu.VMEM((K, C), jnp.int32),
        pltpu.VMEM((C, D), _f32),
        pltpu.VMEM((C, D), _f32),
        pltpu.VMEM_SHARED((NP, D), _f32),
        pltpu.SemaphoreType.DMA,
    ]
    return pl.kernel(
        functools.partial(_seg_body, n_types),
        out_type=out_type,
        mesh=plsc.VectorSubcoreMesh(core_axis_name="c", subcore_axis_name="s"),
        scratch_types=scratch,
    )


HR = NP // D
GC = 5120 // 16


def _cnt_body(d03, d33, d30, z128, C03, C33, C30,
              didx2, hist, idv, acc80, sem):
    core = lax.axis_index("c")
    sub = lax.axis_index("s")
    wid = core * NS + sub
    del sem
    base_iota = lax.iota(jnp.int32, 16)
    for c in range(HR // 16):
        idv[pl.ds(c * 16, 16)] = base_iota + 16 * c

    for dst_h, C_out in ((d03, C03), (d33, C33), (d30, C30)):
        pltpu.sync_copy(dst_h.at[wid], didx2)
        pltpu.sync_copy(z128.at[pl.ds(0, HR)], hist)

        @pl.when(sub == 0)
        def _():
            pltpu.sync_copy(z128.at[pl.ds(0, HR)], acc80)

        plsc.subcore_barrier()

        ones = jnp.ones((16,), _f32)

        def group(g, carry):
            v = didx2[g]
            plsc.addupdate_scatter(
                hist, [lax.shift_right_logical(v, 7), lax.bitwise_and(v, 127)],
                ones)
            return carry

        lax.fori_loop(0, GC, group, 0)
        pltpu.sync_copy(hist, acc80.at[idv], add=True)
        plsc.subcore_barrier()

        @pl.when(sub == 0)
        def _():
            pltpu.sync_copy(acc80, C_out.at[pl.ds(core * HR, HR)])

        plsc.subcore_barrier()


def _make_cnt_kernel():
    return pl.kernel(
        _cnt_body,
        out_type=[jax.ShapeDtypeStruct((NC * HR, D), _f32)] * 3,
        mesh=plsc.VectorSubcoreMesh(core_axis_name="c", subcore_axis_name="s"),
        compiler_params=pltpu.CompilerParams(needs_layout_passes=False),
        scratch_types=[
            pltpu.VMEM((GC, 16), jnp.int32),
            pltpu.VMEM((HR, D), _f32),
            pltpu.VMEM((HR,), jnp.int32),
            pltpu.VMEM_SHARED((HR, D), _f32),
            pltpu.SemaphoreType.DMA,
        ],
    )


def _leaky(h):
    return jnp.where(h >= 0, h, 0.01 * h)


def _agg(pa, pb, ca, cb):
    cnt = jnp.maximum(ca + cb, 1.0)
    return (pa + pb) / cnt


def _tc0_body(x0r, x3r, p03a, p03b, p33a, p33b, p30a, p30b,
              c03a, c03b, c33a, c33b, c30a, c30b,
              wl03, wr03, bl03, wl33, wr33, bl33, wl30, wr30, bl30,
              o0r, o3r):
    dot = functools.partial(jnp.dot, preferred_element_type=_f32)
    a03 = _agg(p03a[...], p03b[...], c03a[...], c03b[...])
    a33 = _agg(p33a[...], p33b[...], c33a[...], c33b[...])
    a30 = _agg(p30a[...], p30b[...], c30a[...], c30b[...])
    x3v = x3r[...]
    h3 = (dot(a03, wl03[...]) + bl03[...] + dot(x3v, wr03[...])
          + dot(a33, wl33[...]) + bl33[...] + dot(x3v, wr33[...]))
    h0 = dot(a30, wl30[...]) + bl30[...] + dot(x0r[...], wr30[...])
    o3r[...] = _leaky(h3)
    o0r[...] = _leaky(h0)


def _tc1_body(x3r, p03a, p03b, p33a, p33b,
              c03a, c03b, c33a, c33b,
              wl03, wr03, bl03, wl33, wr33, bl33, wlin, blin,
              outr):
    dot = functools.partial(jnp.dot, preferred_element_type=_f32)
    a03 = _agg(p03a[...], p03b[...], c03a[...], c03b[...])
    a33 = _agg(p33a[...], p33b[...], c33a[...], c33b[...])
    x3v = x3r[...]
    h3 = (dot(a03, wl03[...]) + bl03[...] + dot(x3v, wr03[...])
          + dot(a33, wl33[...]) + bl33[...] + dot(x3v, wr33[...]))
    outr[...] = dot(_leaky(h3), wlin[...]) + blin[...]


_BM = 1000


def _row_spec(w):
    return pl.BlockSpec((_BM, w), lambda i: (i, 0))


def _full_spec(shape):
    return pl.BlockSpec(shape, lambda i: (0,) * len(shape))


def _tc_layer0(x0, x3, parts, cnts, W):
    (p03a, p03b), (p33a, p33b), (p30a, p30b) = parts
    (c03a, c03b), (c33a, c33b), (c30a, c30b) = cnts
    in_specs = ([_row_spec(D)] * 8 + [_row_spec(1)] * 6
                + [_full_spec((D, D)), _full_spec((D, D)), _full_spec((1, D))] * 3)
    out_specs = [_row_spec(D), _row_spec(D)]
    f = pl.pallas_call(
        _tc0_body,
        grid=(N // _BM,),
        in_specs=in_specs,
        out_specs=out_specs,
        out_shape=[jax.ShapeDtypeStruct((N, D), _f32)] * 2,
    )
    return f(x0, x3, p03a, p03b, p33a, p33b, p30a, p30b,
             c03a, c03b, c33a, c33b, c30a, c30b, *W)


def _tc_layer1(x3, parts, cnts, W):
    (p03a, p03b), (p33a, p33b) = parts
    (c03a, c03b), (c33a, c33b) = cnts
    in_specs = ([_row_spec(D)] * 5 + [_row_spec(1)] * 4
                + [_full_spec((D, D)), _full_spec((D, D)), _full_spec((1, D))] * 2
                + [_full_spec((D, OUT)), _full_spec((1, OUT))])
    f = pl.pallas_call(
        _tc1_body,
        grid=(N // _BM,),
        in_specs=in_specs,
        out_specs=_row_spec(OUT),
        out_shape=jax.ShapeDtypeStruct((N, OUT), _f32),
    )
    return f(x3, p03a, p03b, p33a, p33b, c03a, c03b, c33a, c33b, *W)


def kernel(x_0, x_3, edge_index_0_to_3, edge_index_3_to_0, edge_index_3_to_3,
           Wl_0_e03, bl_0_e03, Wr_0_e03,
           Wl_0_e30, bl_0_e30, Wr_0_e30,
           Wl_0_e33, bl_0_e33, Wr_0_e33,
           Wl_1_e03, bl_1_e03, Wr_1_e03,
           Wl_1_e30, bl_1_e30, Wr_1_e30,
           Wl_1_e33, bl_1_e33, Wr_1_e33,
           W_lin, b_lin):
    s03 = edge_index_0_to_3[0].reshape(NW, K, C)
    d03 = edge_index_0_to_3[1].reshape(NW, K, C)
    s33 = edge_index_3_to_3[0].reshape(NW, K, C)
    d33 = edge_index_3_to_3[1].reshape(NW, K, C)
    s30 = edge_index_3_to_0[0].reshape(NW, K, C)
    d30 = edge_index_3_to_0[1].reshape(NW, K, C)
    z128 = jnp.zeros((R, D), _f32)

    seg3 = _make_seg_kernel(3)
    P03, P33, P30 = seg3(x_0, x_3, s03, d03, s33, d33, s30, d30, z128)

    def padgrp(e):
        f = e.reshape(NW, EPW)
        pad = jnp.full((NW, GC * 16 - EPW), NP - 1, jnp.int32)
        return jnp.concatenate([f, pad], axis=1).reshape(NW, GC, 16)

    C03, C33, C30 = _make_cnt_kernel()(
        padgrp(edge_index_0_to_3[1]), padgrp(edge_index_3_to_3[1]),
        padgrp(edge_index_3_to_0[1]), z128)

    def split(a):
        return a[:N], a[NP:NP + N]

    def csplit(ct):
        return (ct[:HR].reshape(NP)[:N].reshape(N, 1),
                ct[HR:].reshape(NP)[:N].reshape(N, 1))

    cn03, cn33, cn30 = csplit(C03), csplit(C33), csplit(C30)
    x0b, x3b = _tc_layer0(
        x_0, x_3,
        (split(P03), split(P33), split(P30)),
        (cn03, cn33, cn30),
        (Wl_0_e03, Wr_0_e03, bl_0_e03.reshape(1, D),
         Wl_0_e33, Wr_0_e33, bl_0_e33.reshape(1, D),
         Wl_0_e30, Wr_0_e30, bl_0_e30.reshape(1, D)))

    seg2 = _make_seg_kernel(2)
    Q03, Q33 = seg2(x0b, x3b, s03, d03, s33, d33, z128)

    return _tc_layer1(
        x3b, (split(Q03), split(Q33)), (cn03, cn33),
        (Wl_1_e03, Wr_1_e03, bl_1_e03.reshape(1, D),
         Wl_1_e33, Wr_1_e33, bl_1_e33.reshape(1, D),
         W_lin, b_lin.reshape(1, OUT)))


# speedup vs baseline: 6.6191x; 1.1309x over previous
import functools

import jax
import jax.numpy as jnp
from jax import lax
from jax.experimental import pallas as pl
from jax.experimental.pallas import tpu as pltpu
from jax.experimental.pallas import tpu_sc as plsc

N = 10000
D = 128
OUT = 64
E = 160000

NC = 2
NS = 16
NW = NC * NS
EPW = E // NW
C = 125
K = EPW // C
NP = 10240
R = NP // NS
HR = NP // D
GC = 5120 // 16

_f32 = jnp.float32


def _seg_body(n_types, *refs):
    i = 0
    x0 = refs[i]; x3 = refs[i + 1]; i += 2
    edges = []
    for _ in range(n_types):
        edges.append((refs[i], refs[i + 1])); i += 2
    z128 = refs[i]; i += 1
    P_outs = [(refs[i + 2 * t], refs[i + 2 * t + 1]) for t in range(n_types)]
    i += 2 * n_types
    sidx, didx, rows_a, rows_b, acc, gsem = refs[i:i + 6]; i += 6

    core = lax.axis_index("c")
    sub = lax.axis_index("s")
    wid = core * NS + sub
    stripe = pl.ds(sub * R, R)

    xsrc_for = [x0, x3, x3][:n_types] if n_types == 3 else [x0, x3]

    for t in range(n_types):
        src_h, dst_h = edges[t]
        xsrc = xsrc_for[t]
        pltpu.sync_copy(src_h.at[wid], sidx)
        pltpu.sync_copy(dst_h.at[wid], didx)
        pltpu.sync_copy(z128, acc.at[stripe])
        plsc.subcore_barrier()

        pltpu.async_copy(xsrc.at[sidx.at[0]], rows_a, gsem)

        def pair(j2, carry):
            for b, (cur, oth) in ((0, (rows_a, rows_b)),
                                  (1, (rows_b, rows_a))):
                jj = 2 * j2 + b
                pltpu.make_async_copy(
                    xsrc.at[sidx.at[jj]], cur, gsem).wait()

                @pl.when(jj + 1 < K)
                def _():
                    pltpu.async_copy(xsrc.at[sidx.at[jj + 1]], oth, gsem)

                pltpu.sync_copy(cur, acc.at[didx.at[jj]], add=True)
            return carry

        lax.fori_loop(0, K // 2, pair, 0)
        plsc.subcore_barrier()
        pa, pb = P_outs[t]

        @pl.when(core == 0)
        def _():
            pltpu.sync_copy(acc.at[stripe], pa.at[stripe])

        @pl.when(core == 1)
        def _():
            pltpu.sync_copy(acc.at[stripe], pb.at[stripe])


def _make_seg_kernel(n_types):
    out_type = [jax.ShapeDtypeStruct((NP, D), _f32)
                for _ in range(2 * n_types)]
    scratch = [
        pltpu.VMEM((K, C), jnp.int32),
        pltpu.VMEM((K, C), jnp.int32),
        pltpu.VMEM((C, D), _f32),
        pltpu.VMEM((C, D), _f32),
        pltpu.VMEM_SHARED((NP, D), _f32),
        pltpu.SemaphoreType.DMA,
    ]
    return pl.kernel(
        functools.partial(_seg_body, n_types),
        out_type=out_type,
        mesh=plsc.VectorSubcoreMesh(core_axis_name="c", subcore_axis_name="s"),
        scratch_types=scratch,
    )


def _cnt_body(d03, d33, d30, z128, C03, C33, C30,
              didx2, hist, idv, acc80, sem):
    core = lax.axis_index("c")
    sub = lax.axis_index("s")
    wid = core * NS + sub
    del sem
    base_iota = lax.iota(jnp.int32, 16)
    for c in range(HR // 16):
        idv[pl.ds(c * 16, 16)] = base_iota + 16 * c

    for dst_h, C_out in ((d03, C03), (d33, C33), (d30, C30)):
        pltpu.sync_copy(dst_h.at[wid], didx2)
        pltpu.sync_copy(z128.at[pl.ds(0, HR)], hist)

        @pl.when(sub == 0)
        def _():
            pltpu.sync_copy(z128.at[pl.ds(0, HR)], acc80)

        plsc.subcore_barrier()

        ones = jnp.ones((16,), _f32)

        def group(g, carry):
            for b in range(8):
                v = didx2[g, pl.ds(b * 16, 16)]
                plsc.addupdate_scatter(
                    hist,
                    [lax.shift_right_logical(v, 7), lax.bitwise_and(v, 127)],
                    ones)
            return carry

        lax.fori_loop(0, GC // 8, group, 0)
        pltpu.sync_copy(hist, acc80.at[idv], add=True)
        plsc.subcore_barrier()

        @pl.when(sub == 0)
        def _():
            pltpu.sync_copy(acc80, C_out.at[pl.ds(core * HR, HR)])

        plsc.subcore_barrier()


def _make_cnt_kernel():
    return pl.kernel(
        _cnt_body,
        out_type=[jax.ShapeDtypeStruct((NC * HR, D), _f32)] * 3,
        mesh=plsc.VectorSubcoreMesh(core_axis_name="c", subcore_axis_name="s"),
        compiler_params=pltpu.CompilerParams(needs_layout_passes=False),
        scratch_types=[
            pltpu.VMEM((GC // 8, 128), jnp.int32),
            pltpu.VMEM((HR, D), _f32),
            pltpu.VMEM((HR,), jnp.int32),
            pltpu.VMEM_SHARED((HR, D), _f32),
            pltpu.SemaphoreType.DMA,
        ],
    )


def _leaky(h):
    return jnp.where(h >= 0, h, 0.01 * h)


def _agg(pa, pb, ca, cb):
    cnt = jnp.maximum(ca + cb, 1.0)
    return (pa + pb) / cnt


def _tc0_body(x0r, x3r, p03a, p03b, p33a, p33b, p30a, p30b,
              c03a, c03b, c33a, c33b, c30a, c30b,
              wl03, wr03, bl03, wl33, wr33, bl33, wl30, wr30, bl30,
              o0r, o3r):
    dot = functools.partial(jnp.dot, preferred_element_type=_f32)
    a03 = _agg(p03a[...], p03b[...], c03a[...], c03b[...])
    a33 = _agg(p33a[...], p33b[...], c33a[...], c33b[...])
    a30 = _agg(p30a[...], p30b[...], c30a[...], c30b[...])
    x3v = x3r[...]
    h3 = (dot(a03, wl03[...]) + bl03[...] + dot(x3v, wr03[...])
          + dot(a33, wl33[...]) + bl33[...] + dot(x3v, wr33[...]))
    h0 = dot(a30, wl30[...]) + bl30[...] + dot(x0r[...], wr30[...])
    o3r[...] = _leaky(h3)
    o0r[...] = _leaky(h0)


def _tc1_body(x3r, p03a, p03b, p33a, p33b,
              c03a, c03b, c33a, c33b,
              wl03, wr03, bl03, wl33, wr33, bl33, wlin, blin,
              outr):
    dot = functools.partial(jnp.dot, preferred_element_type=_f32)
    a03 = _agg(p03a[...], p03b[...], c03a[...], c03b[...])
    a33 = _agg(p33a[...], p33b[...], c33a[...], c33b[...])
    x3v = x3r[...]
    h3 = (dot(a03, wl03[...]) + bl03[...] + dot(x3v, wr03[...])
          + dot(a33, wl33[...]) + bl33[...] + dot(x3v, wr33[...]))
    outr[...] = dot(_leaky(h3), wlin[...]) + blin[...]


_BM = 1000


def _row_spec(w):
    return pl.BlockSpec((_BM, w), lambda i: (i, 0))


def _full_spec(shape):
    return pl.BlockSpec(shape, lambda i: (0,) * len(shape))


def _tc_layer0(x0, x3, parts, cnts, W):
    (p03a, p03b), (p33a, p33b), (p30a, p30b) = parts
    (c03a, c03b), (c33a, c33b), (c30a, c30b) = cnts
    in_specs = ([_row_spec(D)] * 8 + [_row_spec(1)] * 6
                + [_full_spec((D, D)), _full_spec((D, D)), _full_spec((1, D))] * 3)
    out_specs = [_row_spec(D), _row_spec(D)]
    f = pl.pallas_call(
        _tc0_body,
        grid=(N // _BM,),
        in_specs=in_specs,
        out_specs=out_specs,
        out_shape=[jax.ShapeDtypeStruct((N, D), _f32)] * 2,
    )
    return f(x0, x3, p03a, p03b, p33a, p33b, p30a, p30b,
             c03a, c03b, c33a, c33b, c30a, c30b, *W)


def _tc_layer1(x3, parts, cnts, W):
    (p03a, p03b), (p33a, p33b) = parts
    (c03a, c03b), (c33a, c33b) = cnts
    in_specs = ([_row_spec(D)] * 5 + [_row_spec(1)] * 4
                + [_full_spec((D, D)), _full_spec((D, D)), _full_spec((1, D))] * 2
                + [_full_spec((D, OUT)), _full_spec((1, OUT))])
    f = pl.pallas_call(
        _tc1_body,
        grid=(N // _BM,),
        in_specs=in_specs,
        out_specs=_row_spec(OUT),
        out_shape=jax.ShapeDtypeStruct((N, OUT), _f32),
    )
    return f(x3, p03a, p03b, p33a, p33b, c03a, c03b, c33a, c33b, *W)


def kernel(x_0, x_3, edge_index_0_to_3, edge_index_3_to_0, edge_index_3_to_3,
           Wl_0_e03, bl_0_e03, Wr_0_e03,
           Wl_0_e30, bl_0_e30, Wr_0_e30,
           Wl_0_e33, bl_0_e33, Wr_0_e33,
           Wl_1_e03, bl_1_e03, Wr_1_e03,
           Wl_1_e30, bl_1_e30, Wr_1_e30,
           Wl_1_e33, bl_1_e33, Wr_1_e33,
           W_lin, b_lin):
    s03 = edge_index_0_to_3[0].reshape(NW, K, C)
    d03 = edge_index_0_to_3[1].reshape(NW, K, C)
    s33 = edge_index_3_to_3[0].reshape(NW, K, C)
    d33 = edge_index_3_to_3[1].reshape(NW, K, C)
    s30 = edge_index_3_to_0[0].reshape(NW, K, C)
    d30 = edge_index_3_to_0[1].reshape(NW, K, C)
    z128 = jnp.zeros((R, D), _f32)

    def padgrp(e):
        f = e.reshape(NW, EPW)
        pad = jnp.full((NW, GC * 16 - EPW), NP - 1, jnp.int32)
        return jnp.concatenate([f, pad], axis=1).reshape(NW, GC // 8, 128)

    seg3 = _make_seg_kernel(3)
    Pa03, Pb03, Pa33, Pb33, Pa30, Pb30 = seg3(
        x_0, x_3, s03, d03, s33, d33, s30, d30, z128)

    C03, C33, C30 = _make_cnt_kernel()(
        padgrp(edge_index_0_to_3[1]), padgrp(edge_index_3_to_3[1]),
        padgrp(edge_index_3_to_0[1]), z128)

    def csplit(ct):
        return (ct[:HR].reshape(NP)[:N].reshape(N, 1),
                ct[HR:].reshape(NP)[:N].reshape(N, 1))

    cn03, cn33, cn30 = csplit(C03), csplit(C33), csplit(C30)
    x0b, x3b = _tc_layer0(
        x_0, x_3,
        ((Pa03, Pb03), (Pa33, Pb33), (Pa30, Pb30)),
        (cn03, cn33, cn30),
        (Wl_0_e03, Wr_0_e03, bl_0_e03.reshape(1, D),
         Wl_0_e33, Wr_0_e33, bl_0_e33.reshape(1, D),
         Wl_0_e30, Wr_0_e30, bl_0_e30.reshape(1, D)))

    seg2 = _make_seg_kernel(2)
    Qa03, Qb03, Qa33, Qb33 = seg2(x0b, x3b, s03, d03, s33, d33, z128)

    return _tc_layer1(
        x3b, ((Qa03, Qb03), (Qa33, Qb33)), (cn03, cn33),
        (Wl_1_e03, Wr_1_e03, bl_1_e03.reshape(1, D),
         Wl_1_e33, Wr_1_e33, bl_1_e33.reshape(1, D),
         W_lin, b_lin.reshape(1, OUT)))


# speedup vs baseline: 7.0369x; 1.0631x over previous
import functools

import jax
import jax.numpy as jnp
from jax import lax
from jax.experimental import pallas as pl
from jax.experimental.pallas import tpu as pltpu
from jax.experimental.pallas import tpu_sc as plsc

N = 10000
D = 128
OUT = 64
E = 160000

NC = 2
NS = 16
NW = NC * NS
EPW = E // NW
C = 125
K = EPW // C
NP = 10240
R = NP // NS
HR = NP // D
GC = 5120 // 16

_f32 = jnp.float32


def _scatter_slab(xsrc, s_h, d_h, sid, sidx, didx, rows_a, rows_b, acc, gsem):
    pltpu.sync_copy(s_h.at[sid], sidx)
    pltpu.sync_copy(d_h.at[sid], didx)
    pltpu.async_copy(xsrc.at[sidx.at[0]], rows_a, gsem)

    def pair(j2, carry):
        for b, (cur, oth) in ((0, (rows_a, rows_b)),
                              (1, (rows_b, rows_a))):
            jj = 2 * j2 + b
            pltpu.make_async_copy(xsrc.at[sidx.at[jj]], cur, gsem).wait()

            @pl.when(jj + 1 < K)
            def _():
                pltpu.async_copy(xsrc.at[sidx.at[jj + 1]], oth, gsem)

            pltpu.sync_copy(cur, acc.at[didx.at[jj]], add=True)
        return carry

    lax.fori_loop(0, K // 2, pair, 0)


def _seg_body(dedicated_only, *refs):
    i = 0
    x0 = refs[i]; x3 = refs[i + 1]; i += 2
    s03, d03, s33, d33 = refs[i:i + 4]; i += 4
    if not dedicated_only:
        s30, d30 = refs[i:i + 2]; i += 2
    z128 = refs[i]; i += 1
    P03 = refs[i]; P33 = refs[i + 1]; i += 2
    if not dedicated_only:
        P30a, P30b = refs[i:i + 2]; i += 2
    sidx, didx, rows_a, rows_b, acc, gsem = refs[i:i + 6]; i += 6

    core = lax.axis_index("c")
    sub = lax.axis_index("s")
    wid = core * NS + sub
    stripe = pl.ds(sub * R, R)

    def dedicated(xsrc, s_h, d_h, out):
        pltpu.sync_copy(z128, acc.at[stripe])
        plsc.subcore_barrier()
        _scatter_slab(xsrc, s_h, d_h, sub, sidx, didx,
                      rows_a, rows_b, acc, gsem)
        _scatter_slab(xsrc, s_h, d_h, sub + NS, sidx, didx,
                      rows_a, rows_b, acc, gsem)
        plsc.subcore_barrier()
        pltpu.sync_copy(acc.at[stripe], out.at[stripe])

    @pl.when(core == 0)
    def _():
        dedicated(x0, s03, d03, P03)

    @pl.when(core == 1)
    def _():
        dedicated(x3, s33, d33, P33)

    if dedicated_only:
        return

    pltpu.sync_copy(z128, acc.at[stripe])
    plsc.subcore_barrier()
    _scatter_slab(x3, s30, d30, wid, sidx, didx, rows_a, rows_b, acc, gsem)
    plsc.subcore_barrier()

    @pl.when(core == 0)
    def _():
        pltpu.sync_copy(acc.at[stripe], P30a.at[stripe])

    @pl.when(core == 1)
    def _():
        pltpu.sync_copy(acc.at[stripe], P30b.at[stripe])


def _make_seg_kernel(dedicated_only):
    n_out = 2 if dedicated_only else 4
    out_type = [jax.ShapeDtypeStruct((NP, D), _f32) for _ in range(n_out)]
    scratch = [
        pltpu.VMEM((K, C), jnp.int32),
        pltpu.VMEM((K, C), jnp.int32),
        pltpu.VMEM((C, D), _f32),
        pltpu.VMEM((C, D), _f32),
        pltpu.VMEM_SHARED((NP, D), _f32),
        pltpu.SemaphoreType.DMA,
    ]
    return pl.kernel(
        functools.partial(_seg_body, dedicated_only),
        out_type=out_type,
        mesh=plsc.VectorSubcoreMesh(core_axis_name="c", subcore_axis_name="s"),
        scratch_types=scratch,
    )


def _cnt_body(d03, d33, d30, z128, C03, C33, C30,
              didx2, hist, idv, acc80, sem):
    core = lax.axis_index("c")
    sub = lax.axis_index("s")
    wid = core * NS + sub
    del sem
    base_iota = lax.iota(jnp.int32, 16)
    for c in range(HR // 16):
        idv[pl.ds(c * 16, 16)] = base_iota + 16 * c

    for dst_h, C_out in ((d03, C03), (d33, C33), (d30, C30)):
        pltpu.sync_copy(dst_h.at[wid], didx2)
        pltpu.sync_copy(z128.at[pl.ds(0, HR)], hist)

        @pl.when(sub == 0)
        def _():
            pltpu.sync_copy(z128.at[pl.ds(0, HR)], acc80)

        plsc.subcore_barrier()

        ones = jnp.ones((16,), _f32)

        def group(g, carry):
            for b in range(8):
                v = didx2[g, pl.ds(b * 16, 16)]
                plsc.addupdate_scatter(
                    hist,
                    [lax.shift_right_logical(v, 7), lax.bitwise_and(v, 127)],
                    ones)
            return carry

        lax.fori_loop(0, GC // 8, group, 0)
        pltpu.sync_copy(hist, acc80.at[idv], add=True)
        plsc.subcore_barrier()

        @pl.when(sub == 0)
        def _():
            pltpu.sync_copy(acc80, C_out.at[pl.ds(core * HR, HR)])

        plsc.subcore_barrier()


def _make_cnt_kernel():
    return pl.kernel(
        _cnt_body,
        out_type=[jax.ShapeDtypeStruct((NC * HR, D), _f32)] * 3,
        mesh=plsc.VectorSubcoreMesh(core_axis_name="c", subcore_axis_name="s"),
        compiler_params=pltpu.CompilerParams(needs_layout_passes=False),
        scratch_types=[
            pltpu.VMEM((GC // 8, 128), jnp.int32),
            pltpu.VMEM((HR, D), _f32),
            pltpu.VMEM((HR,), jnp.int32),
            pltpu.VMEM_SHARED((HR, D), _f32),
            pltpu.SemaphoreType.DMA,
        ],
    )


def _leaky(h):
    return jnp.where(h >= 0, h, 0.01 * h)


def _agg(p, ca, cb):
    return p / jnp.maximum(ca + cb, 1.0)


def _tc0_body(x0r, x3r, p03, p33, p30a, p30b,
              c03a, c03b, c33a, c33b, c30a, c30b,
              wl03, wr03, bl03, wl33, wr33, bl33, wl30, wr30, bl30,
              o0r, o3r):
    dot = functools.partial(jnp.dot, preferred_element_type=_f32)
    a03 = _agg(p03[...], c03a[...], c03b[...])
    a33 = _agg(p33[...], c33a[...], c33b[...])
    a30 = _agg(p30a[...] + p30b[...], c30a[...], c30b[...])
    x3v = x3r[...]
    h3 = (dot(a03, wl03[...]) + bl03[...] + dot(x3v, wr03[...])
          + dot(a33, wl33[...]) + bl33[...] + dot(x3v, wr33[...]))
    h0 = dot(a30, wl30[...]) + bl30[...] + dot(x0r[...], wr30[...])
    o3r[...] = _leaky(h3)
    o0r[...] = _leaky(h0)


def _tc1_body(x3r, p03, p33,
              c03a, c03b, c33a, c33b,
              wl03, wr03, bl03, wl33, wr33, bl33, wlin, blin,
              outr):
    dot = functools.partial(jnp.dot, preferred_element_type=_f32)
    a03 = _agg(p03[...], c03a[...], c03b[...])
    a33 = _agg(p33[...], c33a[...], c33b[...])
    x3v = x3r[...]
    h3 = (dot(a03, wl03[...]) + bl03[...] + dot(x3v, wr03[...])
          + dot(a33, wl33[...]) + bl33[...] + dot(x3v, wr33[...]))
    outr[...] = dot(_leaky(h3), wlin[...]) + blin[...]


_BM = 1000


def _row_spec(w):
    return pl.BlockSpec((_BM, w), lambda i: (i, 0))


def _full_spec(shape):
    return pl.BlockSpec(shape, lambda i: (0,) * len(shape))


def _tc_layer0(x0, x3, p03, p33, p30a, p30b, cnts, W):
    (c03a, c03b), (c33a, c33b), (c30a, c30b) = cnts
    in_specs = ([_row_spec(D)] * 6 + [_row_spec(1)] * 6
                + [_full_spec((D, D)), _full_spec((D, D)), _full_spec((1, D))] * 3)
    out_specs = [_row_spec(D), _row_spec(D)]
    f = pl.pallas_call(
        _tc0_body,
        grid=(N // _BM,),
        in_specs=in_specs,
        out_specs=out_specs,
        out_shape=[jax.ShapeDtypeStruct((N, D), _f32)] * 2,
    )
    return f(x0, x3, p03, p33, p30a, p30b,
             c03a, c03b, c33a, c33b, c30a, c30b, *W)


def _tc_layer1(x3, p03, p33, cnts, W):
    (c03a, c03b), (c33a, c33b) = cnts
    in_specs = ([_row_spec(D)] * 3 + [_row_spec(1)] * 4
                + [_full_spec((D, D)), _full_spec((D, D)), _full_spec((1, D))] * 2
                + [_full_spec((D, OUT)), _full_spec((1, OUT))])
    f = pl.pallas_call(
        _tc1_body,
        grid=(N // _BM,),
        in_specs=in_specs,
        out_specs=_row_spec(OUT),
        out_shape=jax.ShapeDtypeStruct((N, OUT), _f32),
    )
    return f(x3, p03, p33, c03a, c03b, c33a, c33b, *W)


def kernel(x_0, x_3, edge_index_0_to_3, edge_index_3_to_0, edge_index_3_to_3,
           Wl_0_e03, bl_0_e03, Wr_0_e03,
           Wl_0_e30, bl_0_e30, Wr_0_e30,
           Wl_0_e33, bl_0_e33, Wr_0_e33,
           Wl_1_e03, bl_1_e03, Wr_1_e03,
           Wl_1_e30, bl_1_e30, Wr_1_e30,
           Wl_1_e33, bl_1_e33, Wr_1_e33,
           W_lin, b_lin):
    s03 = edge_index_0_to_3[0].reshape(NW, K, C)
    d03 = edge_index_0_to_3[1].reshape(NW, K, C)
    s33 = edge_index_3_to_3[0].reshape(NW, K, C)
    d33 = edge_index_3_to_3[1].reshape(NW, K, C)
    s30 = edge_index_3_to_0[0].reshape(NW, K, C)
    d30 = edge_index_3_to_0[1].reshape(NW, K, C)
    z128 = jnp.zeros((R, D), _f32)

    def padgrp(e):
        f = e.reshape(NW, EPW)
        pad = jnp.full((NW, GC * 16 - EPW), NP - 1, jnp.int32)
        return jnp.concatenate([f, pad], axis=1).reshape(NW, GC // 8, 128)

    P03, P33, P30a, P30b = _make_seg_kernel(False)(
        x_0, x_3, s03, d03, s33, d33, s30, d30, z128)

    C03, C33, C30 = _make_cnt_kernel()(
        padgrp(edge_index_0_to_3[1]), padgrp(edge_index_3_to_3[1]),
        padgrp(edge_index_3_to_0[1]), z128)

    def csplit(ct):
        return (ct[:HR].reshape(NP)[:N].reshape(N, 1),
                ct[HR:].reshape(NP)[:N].reshape(N, 1))

    cn03, cn33, cn30 = csplit(C03), csplit(C33), csplit(C30)
    x0b, x3b = _tc_layer0(
        x_0, x_3, P03, P33, P30a, P30b,
        (cn03, cn33, cn30),
        (Wl_0_e03, Wr_0_e03, bl_0_e03.reshape(1, D),
         Wl_0_e33, Wr_0_e33, bl_0_e33.reshape(1, D),
         Wl_0_e30, Wr_0_e30, bl_0_e30.reshape(1, D)))

    Q03, Q33 = _make_seg_kernel(True)(x0b, x3b, s03, d03, s33, d33, z128)

    return _tc_layer1(
        x3b, Q03, Q33, (cn03, cn33),
        (Wl_1_e03, Wr_1_e03, bl_1_e03.reshape(1, D),
         Wl_1_e33, Wr_1_e33, bl_1_e33.reshape(1, D),
         W_lin, b_lin.reshape(1, OUT)))


# speedup vs baseline: 7.2026x; 1.0235x over previous
import functools

import jax
import jax.numpy as jnp
from jax import lax
from jax.experimental import pallas as pl
from jax.experimental.pallas import tpu as pltpu
from jax.experimental.pallas import tpu_sc as plsc

N = 10000
D = 128
OUT = 64
E = 160000

NC = 2
NS = 16
NW = NC * NS
EPW = E // NW
C = 125
K = EPW // C
NP = 10240
R = NP // NS
HR = NP // D
GC = 5120 // 16

_f32 = jnp.float32


def _load_and_prime(xsrc, s_h, d_h, sid, sidx, didx, rows_a, gsem):
    pltpu.sync_copy(s_h.at[sid], sidx)
    pltpu.sync_copy(d_h.at[sid], didx)
    pltpu.async_copy(xsrc.at[sidx.at[0]], rows_a, gsem)


def _chunk_loop(xsrc, sidx, didx, rows_a, rows_b, acc, gsem):
    def pair(j2, carry):
        for b, (cur, oth) in ((0, (rows_a, rows_b)),
                              (1, (rows_b, rows_a))):
            jj = 2 * j2 + b
            pltpu.make_async_copy(xsrc.at[sidx.at[jj]], cur, gsem).wait()

            @pl.when(jj + 1 < K)
            def _():
                pltpu.async_copy(xsrc.at[sidx.at[jj + 1]], oth, gsem)

            pltpu.sync_copy(cur, acc.at[didx.at[jj]], add=True)
        return carry

    lax.fori_loop(0, K // 2, pair, 0)


def _seg_body(dedicated_only, *refs):
    i = 0
    x0 = refs[i]; x3 = refs[i + 1]; i += 2
    s03, d03, s33, d33 = refs[i:i + 4]; i += 4
    if not dedicated_only:
        s30, d30 = refs[i:i + 2]; i += 2
    z128 = refs[i]; i += 1
    P03 = refs[i]; P33 = refs[i + 1]; i += 2
    if not dedicated_only:
        P30a, P30b = refs[i:i + 2]; i += 2
    sidx, didx, rows_a, rows_b, acc, gsem, zsem = refs[i:i + 7]; i += 7

    core = lax.axis_index("c")
    sub = lax.axis_index("s")
    wid = core * NS + sub
    stripe = pl.ds(sub * R, R)

    def start_pass(xsrc, s_h, d_h, sid):
        pltpu.async_copy(z128, acc.at[stripe], zsem)
        _load_and_prime(xsrc, s_h, d_h, sid, sidx, didx, rows_a, gsem)
        pltpu.make_async_copy(z128, acc.at[stripe], zsem).wait()
        plsc.subcore_barrier()

    def dedicated(xsrc, s_h, d_h, out):
        start_pass(xsrc, s_h, d_h, sub)
        _chunk_loop(xsrc, sidx, didx, rows_a, rows_b, acc, gsem)
        _load_and_prime(xsrc, s_h, d_h, sub + NS, sidx, didx, rows_a, gsem)
        _chunk_loop(xsrc, sidx, didx, rows_a, rows_b, acc, gsem)
        plsc.subcore_barrier()
        pltpu.sync_copy(acc.at[stripe], out.at[stripe])

    @pl.when(core == 0)
    def _():
        dedicated(x0, s03, d03, P03)

    @pl.when(core == 1)
    def _():
        dedicated(x3, s33, d33, P33)

    if dedicated_only:
        return

    start_pass(x3, s30, d30, wid)
    _chunk_loop(x3, sidx, didx, rows_a, rows_b, acc, gsem)
    plsc.subcore_barrier()

    @pl.when(core == 0)
    def _():
        pltpu.sync_copy(acc.at[stripe], P30a.at[stripe])

    @pl.when(core == 1)
    def _():
        pltpu.sync_copy(acc.at[stripe], P30b.at[stripe])


def _make_seg_kernel(dedicated_only):
    n_out = 2 if dedicated_only else 4
    out_type = [jax.ShapeDtypeStruct((NP, D), _f32) for _ in range(n_out)]
    scratch = [
        pltpu.VMEM((K, C), jnp.int32),
        pltpu.VMEM((K, C), jnp.int32),
        pltpu.VMEM((C, D), _f32),
        pltpu.VMEM((C, D), _f32),
        pltpu.VMEM_SHARED((NP, D), _f32),
        pltpu.SemaphoreType.DMA,
        pltpu.SemaphoreType.DMA,
    ]
    return pl.kernel(
        functools.partial(_seg_body, dedicated_only),
        out_type=out_type,
        mesh=plsc.VectorSubcoreMesh(core_axis_name="c", subcore_axis_name="s"),
        scratch_types=scratch,
    )


def _cnt_body(d03, d33, d30, z128, C03, C33, C30,
              didx2, hist, idv, acc80, sem):
    core = lax.axis_index("c")
    sub = lax.axis_index("s")
    wid = core * NS + sub
    del sem
    base_iota = lax.iota(jnp.int32, 16)
    for c in range(HR // 16):
        idv[pl.ds(c * 16, 16)] = base_iota + 16 * c

    for dst_h, C_out in ((d03, C03), (d33, C33), (d30, C30)):
        pltpu.sync_copy(dst_h.at[wid], didx2)
        pltpu.sync_copy(z128.at[pl.ds(0, HR)], hist)

        @pl.when(sub == 0)
        def _():
            pltpu.sync_copy(z128.at[pl.ds(0, HR)], acc80)

        plsc.subcore_barrier()

        ones = jnp.ones((16,), _f32)

        def group(g, carry):
            for b in range(8):
                v = didx2[g, pl.ds(b * 16, 16)]
                plsc.addupdate_scatter(
                    hist,
                    [lax.shift_right_logical(v, 7), lax.bitwise_and(v, 127)],
                    ones)
            return carry

        lax.fori_loop(0, GC // 8, group, 0)
        pltpu.sync_copy(hist, acc80.at[idv], add=True)
        plsc.subcore_barrier()

        @pl.when(sub == 0)
        def _():
            pltpu.sync_copy(acc80, C_out.at[pl.ds(core * HR, HR)])

        plsc.subcore_barrier()


def _make_cnt_kernel():
    return pl.kernel(
        _cnt_body,
        out_type=[jax.ShapeDtypeStruct((NC * HR, D), _f32)] * 3,
        mesh=plsc.VectorSubcoreMesh(core_axis_name="c", subcore_axis_name="s"),
        compiler_params=pltpu.CompilerParams(needs_layout_passes=False),
        scratch_types=[
            pltpu.VMEM((GC // 8, 128), jnp.int32),
            pltpu.VMEM((HR, D), _f32),
            pltpu.VMEM((HR,), jnp.int32),
            pltpu.VMEM_SHARED((HR, D), _f32),
            pltpu.SemaphoreType.DMA,
        ],
    )


def _leaky(h):
    return jnp.where(h >= 0, h, 0.01 * h)


def _agg(p, ca, cb):
    return p / jnp.maximum(ca + cb, 1.0)


def _tc0_body(x0r, x3r, p03, p33, p30a, p30b,
              c03a, c03b, c33a, c33b, c30a, c30b,
              wl03, wr03, bl03, wl33, wr33, bl33, wl30, wr30, bl30,
              o0r, o3r):
    dot = functools.partial(jnp.dot, preferred_element_type=_f32)
    a03 = _agg(p03[...], c03a[...], c03b[...])
    a33 = _agg(p33[...], c33a[...], c33b[...])
    a30 = _agg(p30a[...] + p30b[...], c30a[...], c30b[...])
    x3v = x3r[...]
    h3 = (dot(a03, wl03[...]) + bl03[...] + dot(x3v, wr03[...])
          + dot(a33, wl33[...]) + bl33[...] + dot(x3v, wr33[...]))
    h0 = dot(a30, wl30[...]) + bl30[...] + dot(x0r[...], wr30[...])
    o3r[...] = _leaky(h3)
    o0r[...] = _leaky(h0)


def _tc1_body(x3r, p03, p33,
              c03a, c03b, c33a, c33b,
              wl03, wr03, bl03, wl33, wr33, bl33, wlin, blin,
              outr):
    dot = functools.partial(jnp.dot, preferred_element_type=_f32)
    a03 = _agg(p03[...], c03a[...], c03b[...])
    a33 = _agg(p33[...], c33a[...], c33b[...])
    x3v = x3r[...]
    h3 = (dot(a03, wl03[...]) + bl03[...] + dot(x3v, wr03[...])
          + dot(a33, wl33[...]) + bl33[...] + dot(x3v, wr33[...]))
    outr[...] = dot(_leaky(h3), wlin[...]) + blin[...]


_BM = 2000


def _row_spec(w):
    return pl.BlockSpec((_BM, w), lambda i: (i, 0))


def _full_spec(shape):
    return pl.BlockSpec(shape, lambda i: (0,) * len(shape))


def _tc_layer0(x0, x3, p03, p33, p30a, p30b, cnts, W):
    (c03a, c03b), (c33a, c33b), (c30a, c30b) = cnts
    in_specs = ([_row_spec(D)] * 6 + [_row_spec(1)] * 6
                + [_full_spec((D, D)), _full_spec((D, D)), _full_spec((1, D))] * 3)
    out_specs = [_row_spec(D), _row_spec(D)]
    f = pl.pallas_call(
        _tc0_body,
        grid=(N // _BM,),
        in_specs=in_specs,
        out_specs=out_specs,
        out_shape=[jax.ShapeDtypeStruct((N, D), _f32)] * 2,
    )
    return f(x0, x3, p03, p33, p30a, p30b,
             c03a, c03b, c33a, c33b, c30a, c30b, *W)


def _tc_layer1(x3, p03, p33, cnts, W):
    (c03a, c03b), (c33a, c33b) = cnts
    in_specs = ([_row_spec(D)] * 3 + [_row_spec(1)] * 4
                + [_full_spec((D, D)), _full_spec((D, D)), _full_spec((1, D))] * 2
                + [_full_spec((D, OUT)), _full_spec((1, OUT))])
    f = pl.pallas_call(
        _tc1_body,
        grid=(N // _BM,),
        in_specs=in_specs,
        out_specs=_row_spec(OUT),
        out_shape=jax.ShapeDtypeStruct((N, OUT), _f32),
    )
    return f(x3, p03, p33, c03a, c03b, c33a, c33b, *W)


def kernel(x_0, x_3, edge_index_0_to_3, edge_index_3_to_0, edge_index_3_to_3,
           Wl_0_e03, bl_0_e03, Wr_0_e03,
           Wl_0_e30, bl_0_e30, Wr_0_e30,
           Wl_0_e33, bl_0_e33, Wr_0_e33,
           Wl_1_e03, bl_1_e03, Wr_1_e03,
           Wl_1_e30, bl_1_e30, Wr_1_e30,
           Wl_1_e33, bl_1_e33, Wr_1_e33,
           W_lin, b_lin):
    s03 = edge_index_0_to_3[0].reshape(NW, K, C)
    d03 = edge_index_0_to_3[1].reshape(NW, K, C)
    s33 = edge_index_3_to_3[0].reshape(NW, K, C)
    d33 = edge_index_3_to_3[1].reshape(NW, K, C)
    s30 = edge_index_3_to_0[0].reshape(NW, K, C)
    d30 = edge_index_3_to_0[1].reshape(NW, K, C)
    z128 = jnp.zeros((R, D), _f32)

    def padgrp(e):
        f = e.reshape(NW, EPW)
        pad = jnp.full((NW, GC * 16 - EPW), NP - 1, jnp.int32)
        return jnp.concatenate([f, pad], axis=1).reshape(NW, GC // 8, 128)

    P03, P33, P30a, P30b = _make_seg_kernel(False)(
        x_0, x_3, s03, d03, s33, d33, s30, d30, z128)

    C03, C33, C30 = _make_cnt_kernel()(
        padgrp(edge_index_0_to_3[1]), padgrp(edge_index_3_to_3[1]),
        padgrp(edge_index_3_to_0[1]), z128)

    def csplit(ct):
        return (ct[:HR].reshape(NP)[:N].reshape(N, 1),
                ct[HR:].reshape(NP)[:N].reshape(N, 1))

    cn03, cn33, cn30 = csplit(C03), csplit(C33), csplit(C30)
    x0b, x3b = _tc_layer0(
        x_0, x_3, P03, P33, P30a, P30b,
        (cn03, cn33, cn30),
        (Wl_0_e03, Wr_0_e03, bl_0_e03.reshape(1, D),
         Wl_0_e33, Wr_0_e33, bl_0_e33.reshape(1, D),
         Wl_0_e30, Wr_0_e30, bl_0_e30.reshape(1, D)))

    Q03, Q33 = _make_seg_kernel(True)(x0b, x3b, s03, d03, s33, d33, z128)

    return _tc_layer1(
        x3b, Q03, Q33, (cn03, cn33),
        (Wl_1_e03, Wr_1_e03, bl_1_e03.reshape(1, D),
         Wl_1_e33, Wr_1_e33, bl_1_e33.reshape(1, D),
         W_lin, b_lin.reshape(1, OUT)))
